# baseline (speedup 1.0000x reference)
import jax
import jax.numpy as jnp
from jax import lax
from jax.experimental import pallas as pl
from jax.experimental.pallas import tpu as pltpu
from jax.experimental.pallas import tpu_sc as plsc

_NC = 2
_NS = 16
_NW = _NC * _NS
_L = 16

_GE = 40
_GI = 2 * _GE
_CG = 10
_CE = _GE * _CG


def _proj_body(mvc_ref, w_ref, b_ref, out_ref):
    out_ref[...] = (
        jnp.dot(mvc_ref[...], w_ref[...], preferred_element_type=jnp.float32)
        + b_ref[...]
    )


def _node_projection(mvc, wcat, bcat):
    n, d = mvc.shape
    w = wcat.shape[1]
    blk = 1000
    return pl.pallas_call(
        _proj_body,
        grid=(n // blk,),
        in_specs=[
            pl.BlockSpec((blk, d), lambda i: (i, 0)),
            pl.BlockSpec((d, w), lambda i: (0, 0)),
            pl.BlockSpec((1, w), lambda i: (0, 0)),
        ],
        out_specs=pl.BlockSpec((blk, w), lambda i: (i, 0)),
        out_shape=jax.ShapeDtypeStruct((n, w), jnp.float32),
    )(mvc, wcat, bcat)


def _edge_score_sc(table, idx3d, w2r, b2b, n_edges, hidden):
    ew = n_edges // _NW
    n_chunks = ew // _CE
    blocks = _CE // _L

    mesh = plsc.VectorSubcoreMesh(core_axis_name="c", subcore_axis_name="s")

    @pl.kernel(
        out_type=[
            jax.ShapeDtypeStruct((n_edges,), jnp.float32),
            jax.ShapeDtypeStruct((n_edges,), jnp.float32),
        ],
        mesh=mesh,
        compiler_params=pltpu.CompilerParams(
            use_tc_tiling_on_sc=False, needs_layout_passes=False
        ),
        scratch_types=[
            pltpu.VMEM((_CG, _GI), jnp.int32),
            pltpu.VMEM((2 * _CE, hidden), jnp.float32),
            pltpu.VMEM((2, _CE), jnp.float32),
            pltpu.VMEM((2, hidden, _L), jnp.float32),
            pltpu.VMEM((2, _L), jnp.float32),
            pltpu.SemaphoreType.DMA,
        ],
    )
    def k(t_hbm, idx_hbm, w2_hbm, b2_hbm, out0_hbm, out1_hbm,
          idx_v, s_v, o_v, w2_v, b2_v, sem):
        wid = lax.axis_index("s") * _NC + lax.axis_index("c")
        pltpu.sync_copy(w2_hbm, w2_v)
        pltpu.sync_copy(b2_hbm, b2_v)
        iota = lax.iota(jnp.int32, _L)

        def chunk_body(c, _):
            base_e = wid * ew + c * _CE
            pltpu.sync_copy(idx_hbm.at[wid * n_chunks + c], idx_v)
            copies = []
            for g in range(_CG):
                copies.append(
                    pltpu.async_copy(
                        t_hbm.at[idx_v.at[g]],
                        s_v.at[pl.ds(g * _GI, _GI)],
                        sem,
                    )
                )
            for cp in copies:
                cp.wait()

            @plsc.parallel_loop(0, blocks, 1, unroll=2)
            def block_body(b):
                rows_e = 2 * (_L * b + iota)
                rows_o = rows_e + 1
                acc = [b2_v[0, :], jnp.zeros((_L,), jnp.float32),
                       b2_v[1, :], jnp.zeros((_L,), jnp.float32)]
                for j in range(hidden):
                    col = (iota + j) % hidden
                    p = plsc.load_gather(s_v, [rows_e, col])
                    q = plsc.load_gather(s_v, [rows_o, col])
                    r = jnp.maximum(p + q, 0.0)
                    par = j & 1
                    acc[par] = acc[par] + r * w2_v[0, j, :]
                    acc[2 + par] = acc[2 + par] + r * w2_v[1, j, :]
                o_v[0, pl.ds(b * _L, _L)] = acc[0] + acc[1]
                o_v[1, pl.ds(b * _L, _L)] = acc[2] + acc[3]

            del block_body
            pltpu.sync_copy(o_v.at[0], out0_hbm.at[pl.ds(base_e, _CE)])
            pltpu.sync_copy(o_v.at[1], out1_hbm.at[pl.ds(base_e, _CE)])
            return 0

        lax.fori_loop(0, n_chunks, chunk_body, 0)

    return k(table, idx3d, w2r, b2b)


def kernel(mvc, edge_index, slow_edge_mask, W1, b1, W2, b2):
    n_nodes, d_feat = mvc.shape
    n_edges = edge_index.shape[1]
    hidden = W1.shape[0]

    ei = jnp.where(~slow_edge_mask, edge_index, 0)

    offs = jnp.array([[0], [1]], dtype=jnp.int32)
    flat_idx = (2 * ei + offs).T.reshape(-1)
    idx3d = flat_idx.reshape(-1, _CG, _GI)

    wcat = jnp.concatenate([W1[:, :d_feat].T, W1[:, d_feat:].T], axis=1)
    bcat = jnp.concatenate([b1, jnp.zeros((hidden,), jnp.float32)])[None, :]
    r_nodes = _node_projection(mvc, wcat, bcat)
    table = r_nodes.reshape(2 * n_nodes, hidden)

    jr = (jnp.arange(hidden)[:, None] + jnp.arange(_L)[None, :]) % hidden
    w2r = W2[:, jr].astype(jnp.float32)
    b2b = jnp.broadcast_to(b2[:, None], (2, _L)).astype(jnp.float32)

    out0, out1 = _edge_score_sc(table, idx3d, w2r, b2b, n_edges, hidden)
    return jnp.stack([out0, out1], axis=1)
